# Initial kernel scaffold; baseline (speedup 1.0000x reference)
import jax
import jax.numpy as jnp
from jax.experimental import pallas as pl


def kernel(x, edge_index_0, edge_index_1, W_sage_self, W_sage_neigh, b_sage, W_conv, b_conv, W_gat, a_l, a_r, W_short, b_short, ln_g, ln_b, W_q, W_k, W_v, b_q, b_k, b_v, W_out, b_out):
    raise NotImplementedError("write your pallas kernel here")


# speedup vs baseline: 20.4266x; 20.4266x over previous
import functools

import jax
import jax.numpy as jnp
from jax import lax
from jax.experimental import pallas as pl
from jax.experimental.pallas import tpu as pltpu
from jax.experimental.pallas import tpu_sc as plsc

N = 10000
E = 160000
D = 128
H = 8
HD = D // H
NCLS = 64
LAYERS = 2

NSUB = 16
NPAD = 10240
RPS = NPAD // NSUB
EPW = E // NSUB
CH = 80
NCHUNK = EPW // CH
BLK = 400
GRID = N // BLK

_mesh = plsc.VectorSubcoreMesh(core_axis_name="c", subcore_axis_name="s")
_f32 = jnp.float32


def _fill_rows(ref, nrow, ncol, value):
    v = jnp.full((16,), value, _f32)
    nv = ncol // 16

    def body(k, _):
        i = k // nv
        j = k - i * nv
        ref[i, pl.ds(j * 16, 16)] = v
        return 0

    lax.fori_loop(0, nrow * nv, body, 0)


def _sc_count(i0, i1):
    @functools.partial(
        pl.kernel, mesh=_mesh,
        out_type=jax.ShapeDtypeStruct((2, NPAD, D), _f32),
        scratch_types=[
            pltpu.VMEM((CH,), jnp.int32),
            pltpu.VMEM((CH, D), _f32),
            pltpu.VMEM_SHARED((NPAD, D), _f32),
        ],
    )
    def k(i0r, i1r, out, idxv, ones, acc):
        cid = lax.axis_index("c")
        sid = lax.axis_index("s")

        _fill_rows(ones, CH, D, 0.0)

        def zacc(j, _):
            pltpu.sync_copy(ones, acc.at[pl.ds(sid * RPS + j * CH, CH)])
            return 0

        lax.fori_loop(0, RPS // CH, zacc, 0)
        plsc.subcore_barrier()

        _fill_rows(ones, CH, D, 1.0)

        def count(iref):
            def chunk(j, _):
                base = sid * EPW + j * CH
                pltpu.sync_copy(iref.at[pl.ds(base, CH)], idxv)
                pltpu.sync_copy(ones, acc.at[idxv], add=True)
                return 0

            lax.fori_loop(0, NCHUNK, chunk, 0)

        @pl.when(cid == 0)
        def _():
            count(i0r)

        @pl.when(cid == 1)
        def _():
            count(i1r)

        plsc.subcore_barrier()
        sl = pl.ds(sid * RPS, RPS)
        pltpu.sync_copy(acc.at[sl], out.at[cid, sl])

    return k(i0, i1)


def _sc_segsum(tab0, tab1, s0, d0, s1, d1):
    @functools.partial(
        pl.kernel, mesh=_mesh,
        out_type=jax.ShapeDtypeStruct((2, NPAD, D), _f32),
        scratch_types=[
            pltpu.VMEM((CH,), jnp.int32),
            pltpu.VMEM((CH,), jnp.int32),
            pltpu.VMEM((CH, D), _f32),
            pltpu.SemaphoreType.DMA,
            pltpu.VMEM_SHARED((NPAD, D), _f32),
        ],
    )
    def k(t0, t1, s0r, d0r, s1r, d1r, out, sidx, didx, rows, sem, acc):
        cid = lax.axis_index("c")
        sid = lax.axis_index("s")

        if True:
            _fill_rows(rows, CH, D, 0.0)

            def zacc(j, _):
                pltpu.sync_copy(rows, acc.at[pl.ds(sid * RPS + j * CH, CH)])
                return 0

            lax.fori_loop(0, RPS // CH, zacc, 0)
            plsc.subcore_barrier()

            def accumulate(tref, sref, dref):
                def chunk(j, _):
                    base = sid * EPW + j * CH
                    pltpu.sync_copy(sref.at[pl.ds(base, CH)], sidx)
                    pltpu.async_copy(tref.at[sidx], rows, sem).wait()
                    pltpu.sync_copy(dref.at[pl.ds(base, CH)], didx)
                    pltpu.sync_copy(rows, acc.at[didx], add=True)
                    return 0

                lax.fori_loop(0, NCHUNK, chunk, 0)

            @pl.when(cid == 0)
            def _():
                accumulate(t0, s0r, d0r)

            @pl.when(cid == 1)
            def _():
                accumulate(t1, s1r, d1r)

            plsc.subcore_barrier()
            sl = pl.ds(sid * RPS, RPS)
            pltpu.sync_copy(acc.at[sl], out.at[cid, sl])

    return k(tab0, tab1, s0, d0, s1, d1)


def _sc_gat_pass(hp0, hp1, el0, er0, el1, er1, cmx, s0, d0, s1, d1, with_hp):
    @functools.partial(
        pl.kernel, mesh=_mesh,
        out_type=jax.ShapeDtypeStruct((2, NPAD, D), _f32),
        scratch_types=[
            pltpu.VMEM((CH,), jnp.int32),
            pltpu.VMEM((CH,), jnp.int32),
            pltpu.VMEM((CH, D), _f32),
            pltpu.VMEM((CH, D), _f32),
            pltpu.VMEM((CH, D), _f32),
            pltpu.VMEM((D,), _f32),
            pltpu.SemaphoreType.DMA,
            pltpu.SemaphoreType.DMA,
            pltpu.SemaphoreType.DMA,
            pltpu.VMEM_SHARED((NPAD, D), _f32),
        ],
    )
    def k(hp0r, hp1r, el0r, er0r, el1r, er1r, cmxr,
          s0r, d0r, s1r, d1r, out,
          sidx, didx, rows, elv, erv, cbuf, sem, sem2, sem3, acc):
        cid = lax.axis_index("c")
        sid = lax.axis_index("s")

        _fill_rows(rows, CH, D, 0.0)

        def zacc(j, _):
            pltpu.sync_copy(rows, acc.at[pl.ds(sid * RPS + j * CH, CH)])
            return 0

        lax.fori_loop(0, RPS // CH, zacc, 0)

        pltpu.sync_copy(cmxr.at[cid], cbuf)
        cvec = cbuf[pl.ds(0, 16)]
        c = cvec[0]
        for hh in range(1, H):
            c = jnp.maximum(c, cvec[hh])
        c = jnp.maximum(c, 0.0)
        plsc.subcore_barrier()

        def accumulate(hpr, elr, err, sref, dref):
            def chunk(j, _):
                base = sid * EPW + j * CH
                pltpu.sync_copy(sref.at[pl.ds(base, CH)], sidx)
                pltpu.sync_copy(dref.at[pl.ds(base, CH)], didx)
                pltpu.async_copy(elr.at[sidx], elv, sem).wait()
                pltpu.async_copy(err.at[didx], erv, sem2).wait()
                if with_hp:
                    pltpu.async_copy(hpr.at[sidx], rows, sem3).wait()

                def per_edge(i, _):
                    ve = elv[i, pl.ds(0, 16)] + erv[i, pl.ds(0, 16)]
                    ve = jnp.where(ve > 0, ve, 0.2 * ve)
                    ex = jnp.exp(ve - c)
                    if with_hp:
                        for hh in range(H):
                            sl = pl.ds(hh * HD, HD)
                            rows[i, sl] = rows[i, sl] * ex[hh]
                    else:
                        rows[i, pl.ds(0, 16)] = ex
                    return 0

                lax.fori_loop(0, CH, per_edge, 0)
                pltpu.sync_copy(rows, acc.at[didx], add=True)
                return 0

            lax.fori_loop(0, NCHUNK, chunk, 0)

        @pl.when(cid == 0)
        def _():
            accumulate(hp0r, el0r, er0r, s0r, d0r)

        @pl.when(cid == 1)
        def _():
            accumulate(hp1r, el1r, er1r, s1r, d1r)

        plsc.subcore_barrier()
        sl = pl.ds(sid * RPS, RPS)
        pltpu.sync_copy(acc.at[sl], out.at[cid, sl])

    return k(hp0, hp1, el0, er0, el1, er1, cmx, s0, d0, s1, d1)


def _lnk(x, g, b):
    u = jnp.mean(x, axis=-1, keepdims=True)
    s2 = jnp.mean((x - u) ** 2, axis=-1, keepdims=True)
    return g * (x - u) / jnp.sqrt(s2 + 1e-12) + b


def _eluk(x):
    return jnp.where(x > 0, x, jnp.exp(jnp.minimum(x, 0.0)) - 1.0)


def _head_sum_mat(ncol):
    r = lax.broadcasted_iota(jnp.int32, (D, ncol), 0) // HD
    cc = lax.broadcasted_iota(jnp.int32, (D, ncol), 1)
    return (r == cc).astype(_f32)


def _head_exp_mat():
    rr = lax.broadcasted_iota(jnp.int32, (H, D), 0)
    cc = lax.broadcasted_iota(jnp.int32, (H, D), 1) // HD
    return (rr == cc).astype(_f32)


def _dot(a, b):
    return jnp.dot(a, b, preferred_element_type=_f32)


def _tc_prep(h, dout0, dout1, Wg, al, ar):
    def body(h_ref, do0_ref, do1_ref, wg_ref, al_ref, ar_ref,
             hn0_ref, hn1_ref, hp0_ref, hp1_ref,
             el0_ref, er0_ref, el1_ref, er1_ref, cmx_ref):
        i = pl.program_id(0)
        hb = h_ref[...]
        M128 = _head_sum_mat(D)
        cands = []
        for t in range(2):
            do = (do0_ref if t == 0 else do1_ref)[...][:, 0:1]
            ns = lax.rsqrt(jnp.where(do > 0, do, 1.0))
            (hn0_ref if t == 0 else hn1_ref)[...] = hb * ns
            hp = _dot(hb, wg_ref[t])
            (hp0_ref if t == 0 else hp1_ref)[...] = hp
            el = _dot(hp * al_ref[t], M128)
            er = _dot(hp * ar_ref[t], M128)
            (el0_ref if t == 0 else el1_ref)[...] = el
            (er0_ref if t == 0 else er1_ref)[...] = er
            cands.append((jnp.max(el, axis=0) + jnp.max(er, axis=0))[None, :])
        cand = jnp.concatenate(
            cands + [jnp.zeros((6, D), _f32)], axis=0)
        prev = jnp.maximum(cmx_ref[...], cand)
        cmx_ref[...] = jnp.where(i == 0, cand, prev)

    nspec = pl.BlockSpec((BLK, D), lambda i: (i, 0))
    wspec = pl.BlockSpec((2, D, D), lambda i: (0, 0, 0))
    aspec = pl.BlockSpec((2, 1, D), lambda i: (0, 0, 0))
    return pl.pallas_call(
        body,
        grid=(GRID,),
        in_specs=[nspec, nspec, nspec, wspec, aspec, aspec],
        out_specs=[nspec] * 8 + [pl.BlockSpec((8, D), lambda i: (0, 0))],
        out_shape=[jax.ShapeDtypeStruct((N, D), _f32)] * 8
        + [jax.ShapeDtypeStruct((8, D), _f32)],
    )(h, dout0, dout1, Wg, al, ar)


def _tc_fuse(h, agg0, agg1, hc0, hc1, num0, num1, den0, den1, din0, din1,
             Wss, Wsn, bs, Wc, bc, Wsh, bsh, lng, lnb, Wq, Wk, Wv, bq, bk, bv):
    def body(h_ref, a0_ref, a1_ref, c0_ref, c1_ref, n0_ref, n1_ref,
             de0_ref, de1_ref, di0_ref, di1_ref,
             wss_ref, wsn_ref, bs_ref, wc_ref, bc_ref, wsh_ref, bsh_ref,
             lng_ref, lnb_ref, wq_ref, wk_ref, wv_ref,
             bq_ref, bk_ref, bv_ref, out_ref):
        hb = h_ref[...]
        Mh = _head_sum_mat(H)
        MhT = _head_exp_mat()
        paths = []
        for t in range(2):
            di = (di0_ref if t == 0 else di1_ref)[...][:, 0:1]
            agg = (a0_ref if t == 0 else a1_ref)[...]
            aggn = agg / jnp.maximum(di, 1.0)
            hs = _dot(hb, wss_ref[t]) + _dot(aggn, wsn_ref[t]) + bs_ref[t]
            hs = _eluk(_lnk(hs, lng_ref[0 * 2 + t], lnb_ref[0 * 2 + t]))

            nd = lax.rsqrt(jnp.where(di > 0, di, 1.0))
            hcr = (c0_ref if t == 0 else c1_ref)[...]
            hcv = _dot(hcr * nd, wc_ref[t]) + bc_ref[t]
            hcv = _eluk(_lnk(hcv, lng_ref[1 * 2 + t], lnb_ref[1 * 2 + t]))

            den8 = (de0_ref if t == 0 else de1_ref)[...][:, 0:H]
            denx = _dot(den8, MhT)
            num = (n0_ref if t == 0 else n1_ref)[...]
            ha = num / jnp.maximum(denx, 1e-30)
            ha = _eluk(_lnk(ha, lng_ref[2 * 2 + t], lnb_ref[2 * 2 + t]))

            hr = _dot(hb, wsh_ref[t]) + bsh_ref[t]
            hr = _eluk(_lnk(hr, lng_ref[3 * 2 + t], lnb_ref[3 * 2 + t]))
            paths += [hs, hcv, ha, hr]

        seq = [paths[0], paths[1], paths[2], paths[3],
               paths[4], paths[5], paths[6], paths[7]]
        qs = [_dot(p, wq_ref[...]) + bq_ref[...] for p in seq]
        ks = [_dot(p, wk_ref[...]) + bk_ref[...] for p in seq]
        vs = [_dot(p, wv_ref[...]) + bv_ref[...] for p in seq]

        hout = jnp.zeros_like(hb)
        inv_sqrt_hd = 0.25
        for s in range(8):
            atts = [_dot(qs[s] * ks[t2], Mh) * inv_sqrt_hd for t2 in range(8)]
            amax = atts[0]
            for t2 in range(1, 8):
                amax = jnp.maximum(amax, atts[t2])
            exs = [jnp.exp(a - amax) for a in atts]
            dsum = exs[0]
            for t2 in range(1, 8):
                dsum = dsum + exs[t2]
            ctx = jnp.zeros_like(hb)
            for t2 in range(8):
                w = exs[t2] / dsum
                ctx = ctx + _dot(w, MhT) * vs[t2]
            hout = hout + ctx
        out_ref[...] = hout * 0.125

    nspec = pl.BlockSpec((BLK, D), lambda i: (i, 0))
    w2spec = pl.BlockSpec((2, D, D), lambda i: (0, 0, 0))
    b2spec = pl.BlockSpec((2, 1, D), lambda i: (0, 0, 0))
    l8spec = pl.BlockSpec((8, 1, D), lambda i: (0, 0, 0))
    wspec = pl.BlockSpec((D, D), lambda i: (0, 0))
    bspec = pl.BlockSpec((1, D), lambda i: (0, 0))
    return pl.pallas_call(
        body,
        grid=(GRID,),
        in_specs=[nspec, nspec, nspec, nspec, nspec, nspec, nspec,
                  nspec, nspec, nspec, nspec,
                  w2spec, w2spec, b2spec, w2spec, b2spec, w2spec, b2spec,
                  l8spec, l8spec, wspec, wspec, wspec, bspec, bspec, bspec],
        out_specs=nspec,
        out_shape=jax.ShapeDtypeStruct((N, D), _f32),
    )(h, agg0, agg1, hc0, hc1, num0, num1, den0, den1, din0, din1,
      Wss, Wsn, bs, Wc, bc, Wsh, bsh, lng, lnb, Wq, Wk, Wv, bq, bk, bv)


def _tc_cls(h, Wo, bo):
    def body(h_ref, w_ref, b_ref, out_ref):
        out_ref[...] = _dot(h_ref[...], w_ref[...]) + b_ref[...]

    return pl.pallas_call(
        body,
        grid=(GRID,),
        in_specs=[pl.BlockSpec((BLK, D), lambda i: (i, 0)),
                  pl.BlockSpec((D, NCLS), lambda i: (0, 0)),
                  pl.BlockSpec((1, NCLS), lambda i: (0, 0))],
        out_specs=pl.BlockSpec((BLK, NCLS), lambda i: (i, 0)),
        out_shape=jax.ShapeDtypeStruct((N, NCLS), _f32),
    )(h, Wo, bo)


def kernel(x, edge_index_0, edge_index_1, W_sage_self, W_sage_neigh, b_sage,
           W_conv, b_conv, W_gat, a_l, a_r, W_short, b_short, ln_g, ln_b,
           W_q, W_k, W_v, b_q, b_k, b_v, W_out, b_out):
    s0, d0 = edge_index_0[0], edge_index_0[1]
    s1, d1 = edge_index_1[0], edge_index_1[1]

    din = _sc_count(d0, d1)
    dout = _sc_count(s0, s1)
    din0, din1 = din[0, :N], din[1, :N]
    dout0, dout1 = dout[0, :N], dout[1, :N]

    h = x
    for l in range(LAYERS):
        agg = _sc_segsum(h, h, s0, d0, s1, d1)
        hn0, hn1, hp0, hp1, el0, er0, el1, er1, cmx = _tc_prep(
            h, dout0, dout1, W_gat[l],
            a_l[l].reshape(2, 1, D), a_r[l].reshape(2, 1, D))
        hc = _sc_segsum(hn0, hn1, s0, d0, s1, d1)
        den = _sc_gat_pass(hp0, hp1, el0, er0, el1, er1, cmx,
                           s0, d0, s1, d1, with_hp=False)
        num = _sc_gat_pass(hp0, hp1, el0, er0, el1, er1, cmx,
                           s0, d0, s1, d1, with_hp=True)
        h = _tc_fuse(
            h, agg[0, :N], agg[1, :N], hc[0, :N], hc[1, :N],
            num[0, :N], num[1, :N], den[0, :N], den[1, :N], din0, din1,
            W_sage_self[l], W_sage_neigh[l], b_sage[l].reshape(2, 1, D),
            W_conv[l], b_conv[l].reshape(2, 1, D),
            W_short[l], b_short[l].reshape(2, 1, D),
            ln_g[l].reshape(8, 1, D), ln_b[l].reshape(8, 1, D),
            W_q[l], W_k[l], W_v[l],
            b_q[l].reshape(1, D), b_k[l].reshape(1, D), b_v[l].reshape(1, D))

    return _tc_cls(h, W_out, b_out.reshape(1, NCLS))
